# baseline (speedup 1.0000x reference)
import functools

import jax
import jax.numpy as jnp
from jax import lax
from jax.experimental import pallas as pl
from jax.experimental.pallas import tpu as pltpu
from jax.experimental.pallas import tpu_sc as plsc

_N = 8192
_D = 16
_P = 256
_Q_MIN = 0.01
_REP = 10.0

_NW = 16
_CH = _N // _NW
_PW = _P // _NW
_BIGN = 1 << 30

_N_BLK = 2048
_NB = _N // _N_BLK


@functools.partial(
    pl.kernel,
    mesh=plsc.VectorSubcoreMesh(
        core_axis_name="c", subcore_axis_name="s", num_cores=1),
    compiler_params=pltpu.CompilerParams(needs_layout_passes=False, use_tc_tiling_on_sc=False),
    out_type=[
        jax.ShapeDtypeStruct((_P, _D), jnp.float32),
        jax.ShapeDtypeStruct((_P,), jnp.float32),
    ],
    scratch_types=[
        pltpu.VMEM((_CH,), jnp.float32),
        pltpu.VMEM((_CH,), jnp.int32),
        pltpu.VMEM((_P,), jnp.float32),
        pltpu.VMEM((_P,), jnp.int32),
        pltpu.VMEM((_NW, 32), jnp.int32),
        pltpu.VMEM_SHARED((_NW, _NW, 32), jnp.int32),
        pltpu.VMEM((_NW, 32), jnp.int32),
        pltpu.VMEM((_PW,), jnp.int32),
        pltpu.VMEM((_PW, _D), jnp.float32),
        pltpu.VMEM((_PW,), jnp.float32),
        pltpu.SemaphoreType.DMA,
    ],
)
def _sc_select(beta_hbm, pid_hbm, x_hbm, xa_out, bb_out,
               bbuf, pbuf, tb, tn, pk, sh, mb, idxv, rows, bbv, sem):
    wid = lax.axis_index("s")
    base = wid * _CH
    lane16 = jnp.arange(16, dtype=jnp.int32)

    pltpu.sync_copy(beta_hbm.at[pl.ds(base, _CH)], bbuf)
    pltpu.sync_copy(pid_hbm.at[pl.ds(base, _CH)], pbuf)

    for k in range(_P // 16):
        tb[pl.ds(k * 16, 16)] = jnp.full((16,), -1.0, jnp.float32)
        tn[pl.ds(k * 16, 16)] = jnp.full((16,), _BIGN, jnp.int32)

    for j in range(_CH // 16):
        bvec = bbuf[pl.ds(j * 16, 16)]
        pvec = pbuf[pl.ds(j * 16, 16)]
        gidx = base + j * 16 + lane16
        valid = pvec >= 1
        pm1 = jnp.where(valid, pvec - 1, 0)

        def _want(_, bvec=bvec, gidx=gidx, valid=valid, pm1=pm1):
            cur = plsc.load_gather(tb, [pm1])
            curn = plsc.load_gather(tn, [pm1])
            better = jnp.logical_or(
                bvec > cur,
                jnp.logical_and(bvec == cur, gidx < curn))
            return jnp.logical_and(valid, better)

        def _body(want, bvec=bvec, gidx=gidx, pm1=pm1, _want=_want):
            plsc.store_scatter(tb, [pm1], bvec, mask=want)
            plsc.store_scatter(tn, [pm1], gidx, mask=want)
            return _want(None)

        lax.while_loop(lambda w: jnp.any(w), _body, _want(None))

    for o in range(_NW):
        pk[o, pl.ds(0, 16)] = plsc.bitcast(tb[pl.ds(o * 16, 16)], jnp.int32)
        pk[o, pl.ds(16, 16)] = tn[pl.ds(o * 16, 16)]
    for o in range(_NW):
        pltpu.sync_copy(pk.at[o], sh.at[o, wid])
    plsc.subcore_barrier()

    pltpu.sync_copy(sh.at[wid], mb)

    merged_b = jnp.full((16,), -(2 ** 30), jnp.int32)
    merged_n = jnp.full((16,), _BIGN, jnp.int32)
    for t in range(_NW):
        b_t = mb[t, pl.ds(0, 16)]
        n_t = mb[t, pl.ds(16, 16)]
        better = jnp.logical_or(
            b_t > merged_b,
            jnp.logical_and(b_t == merged_b, n_t < merged_n))
        merged_b = jnp.where(better, b_t, merged_b)
        merged_n = jnp.where(better, n_t, merged_n)

    idxv[...] = jnp.where(merged_b >= 0, merged_n, 0)
    bbv[...] = plsc.bitcast(merged_b, jnp.float32)
    pltpu.async_copy(x_hbm.at[idxv], rows, sem).wait()

    pltpu.sync_copy(rows, xa_out.at[pl.ds(wid * _PW, _PW)])
    pltpu.sync_copy(bbv, bb_out.at[pl.ds(wid * _PW, _PW)])


def _dense_kernel(beta_ref, pid_ref, x_ref, xa_ref, bb_ref, out_ref,
                  racc, tacc):
    b = pl.program_id(0)

    @pl.when(b == 0)
    def _init():
        racc[...] = jnp.zeros((1, _P), jnp.float32)
        tacc[...] = jnp.zeros((1, _P), jnp.float32)

    beta_col = beta_ref[...]
    at = 0.5 * jnp.log((1.0 + beta_col) / (1.0 - beta_col))
    q_col = at * at + _Q_MIN

    x_blk = x_ref[...]
    xa = xa_ref[...]
    xa2 = jnp.sum(xa * xa, axis=1, keepdims=True)
    xaug = jnp.concatenate([-2.0 * xa, xa2], axis=1)
    x_ext = jnp.concatenate(
        [x_blk, jnp.ones((_N_BLK, 1), jnp.float32)], axis=1)

    dot = jax.lax.dot_general(
        x_ext, xaug, (((1,), (1,)), ((), ())),
        preferred_element_type=jnp.float32)
    xn2 = jnp.sum(x_blk * x_blk, axis=1, keepdims=True)
    d2 = jnp.maximum(xn2 + dot, 0.0)
    hinge = jnp.maximum(1.0 - jnp.sqrt(d2), 0.0)

    lane = jax.lax.broadcasted_iota(jnp.int32, (_N_BLK, _P), 1)
    mask = pid_ref[...] == (lane + 1)
    seg = jnp.where(mask, d2 - _REP * hinge, 0.0)

    racc[...] += jax.lax.dot_general(
        q_col, hinge, (((0,), (0,)), ((), ())),
        preferred_element_type=jnp.float32)
    tacc[...] += jax.lax.dot_general(
        q_col, seg, (((0,), (0,)), ((), ())),
        preferred_element_type=jnp.float32)

    @pl.when(b == _NB - 1)
    def _final():
        bb = bb_ref[...]
        present = (bb >= 0.0).astype(jnp.float32)
        bbg = jnp.where(bb >= 0.0, bb, 0.0)
        ata = 0.5 * jnp.log((1.0 + bbg) / (1.0 - bbg))
        qa = ata * ata + _Q_MIN
        s = qa * (tacc[...] + _REP * racc[...]) * present
        out_ref[...] = jnp.sum(s, axis=(0, 1), keepdims=True) / _N


@functools.partial(jax.jit)
def _potential_loss(beta, x, particle_id):
    xa, bb = _sc_select(beta, particle_id, x)

    out = pl.pallas_call(
        _dense_kernel,
        grid=(_NB,),
        in_specs=[
            pl.BlockSpec((_N_BLK, 1), lambda b: (b, 0)),
            pl.BlockSpec((_N_BLK, 1), lambda b: (b, 0)),
            pl.BlockSpec((_N_BLK, _D), lambda b: (b, 0)),
            pl.BlockSpec((_P, _D), lambda b: (0, 0)),
            pl.BlockSpec((1, _P), lambda b: (0, 0)),
        ],
        out_specs=pl.BlockSpec((1, 1), lambda b: (0, 0)),
        out_shape=jax.ShapeDtypeStruct((1, 1), jnp.float32),
        scratch_shapes=[
            pltpu.VMEM((1, _P), jnp.float32),
            pltpu.VMEM((1, _P), jnp.float32),
        ],
        compiler_params=pltpu.CompilerParams(
            dimension_semantics=("arbitrary",),
        ),
    )(beta.reshape(_N, 1), particle_id.reshape(_N, 1), x,
      xa, bb.reshape(1, _P))
    return out[0, 0]


def kernel(w, beta, x, y, particle_id):
    return _potential_loss(beta, x, particle_id)


# speedup vs baseline: 1.0011x; 1.0011x over previous
import functools

import jax
import jax.numpy as jnp
from jax import lax
from jax.experimental import pallas as pl
from jax.experimental.pallas import tpu as pltpu
from jax.experimental.pallas import tpu_sc as plsc

_N = 8192
_D = 16
_P = 256
_Q_MIN = 0.01
_REP = 10.0

_NW = 16
_CH = _N // _NW
_PW = _P // _NW
_BIGN = 1 << 30

_N_BLK = 4096
_NB = _N // _N_BLK


@functools.partial(
    pl.kernel,
    mesh=plsc.VectorSubcoreMesh(
        core_axis_name="c", subcore_axis_name="s", num_cores=1),
    compiler_params=pltpu.CompilerParams(needs_layout_passes=False, use_tc_tiling_on_sc=False),
    out_type=[
        jax.ShapeDtypeStruct((_P, _D), jnp.float32),
        jax.ShapeDtypeStruct((_P,), jnp.float32),
    ],
    scratch_types=[
        pltpu.VMEM((_CH,), jnp.float32),
        pltpu.VMEM((_CH,), jnp.int32),
        pltpu.VMEM((_P,), jnp.float32),
        pltpu.VMEM((_P,), jnp.int32),
        pltpu.VMEM_SHARED((_NW, _P), jnp.float32),
        pltpu.VMEM_SHARED((_NW, _P), jnp.int32),
        pltpu.VMEM((_NW, _P), jnp.float32),
        pltpu.VMEM((_NW, _P), jnp.int32),
        pltpu.VMEM((_PW,), jnp.int32),
        pltpu.VMEM((_PW, _D), jnp.float32),
        pltpu.VMEM((_PW,), jnp.float32),
        pltpu.SemaphoreType.DMA,
    ],
)
def _sc_select(beta_hbm, pid_hbm, x_hbm, xa_out, bb_out,
               bbuf, pbuf, tb, tn, sh_b, sh_n, mb, mn, idxv, rows, bbv, sem):
    wid = lax.axis_index("s")
    base = wid * _CH
    lane16 = jnp.arange(16, dtype=jnp.int32)

    pltpu.sync_copy(beta_hbm.at[pl.ds(base, _CH)], bbuf)
    pltpu.sync_copy(pid_hbm.at[pl.ds(base, _CH)], pbuf)

    for k in range(_P // 16):
        tb[pl.ds(k * 16, 16)] = jnp.full((16,), -1.0, jnp.float32)
        tn[pl.ds(k * 16, 16)] = jnp.full((16,), _BIGN, jnp.int32)

    for j in range(_CH // 16):
        bvec = bbuf[pl.ds(j * 16, 16)]
        pvec = pbuf[pl.ds(j * 16, 16)]
        gidx = base + j * 16 + lane16
        valid = pvec >= 1
        pm1 = jnp.where(valid, pvec - 1, 0)

        def _want(_, bvec=bvec, gidx=gidx, valid=valid, pm1=pm1):
            cur = plsc.load_gather(tb, [pm1])
            curn = plsc.load_gather(tn, [pm1])
            better = jnp.logical_or(
                bvec > cur,
                jnp.logical_and(bvec == cur, gidx < curn))
            return jnp.logical_and(valid, better)

        def _body(want, bvec=bvec, gidx=gidx, pm1=pm1, _want=_want):
            plsc.store_scatter(tb, [pm1], bvec, mask=want)
            plsc.store_scatter(tn, [pm1], gidx, mask=want)
            return _want(None)

        lax.while_loop(lambda w: jnp.any(w), _body, _want(None))

    pltpu.sync_copy(tb, sh_b.at[wid])
    pltpu.sync_copy(tn, sh_n.at[wid])
    plsc.subcore_barrier()

    pltpu.sync_copy(sh_b, mb)
    pltpu.sync_copy(sh_n, mn)

    merged_b = jnp.full((16,), -1.0, jnp.float32)
    merged_n = jnp.full((16,), _BIGN, jnp.int32)
    col = wid * _PW
    for t in range(_NW):
        b_t = mb[t, pl.ds(col, _PW)]
        n_t = mn[t, pl.ds(col, _PW)]
        better = jnp.logical_or(
            b_t > merged_b,
            jnp.logical_and(b_t == merged_b, n_t < merged_n))
        merged_b = jnp.where(better, b_t, merged_b)
        merged_n = jnp.where(better, n_t, merged_n)

    idxv[...] = jnp.where(merged_b >= 0.0, merged_n, 0)
    bbv[...] = merged_b
    pltpu.async_copy(x_hbm.at[idxv], rows, sem).wait()

    pltpu.sync_copy(rows, xa_out.at[pl.ds(wid * _PW, _PW)])
    pltpu.sync_copy(bbv, bb_out.at[pl.ds(wid * _PW, _PW)])


def _dense_kernel(beta_ref, pid_ref, x_ref, xa_ref, bb_ref, out_ref,
                  racc, tacc):
    b = pl.program_id(0)

    @pl.when(b == 0)
    def _init():
        racc[...] = jnp.zeros((1, _P), jnp.float32)
        tacc[...] = jnp.zeros((1, _P), jnp.float32)

    beta_col = beta_ref[...]
    at = 0.5 * jnp.log((1.0 + beta_col) / (1.0 - beta_col))
    q_col = at * at + _Q_MIN

    x_blk = x_ref[...]
    xa = xa_ref[...]
    xa2 = jnp.sum(xa * xa, axis=1, keepdims=True)
    xaug = jnp.concatenate([-2.0 * xa, xa2], axis=1)
    x_ext = jnp.concatenate(
        [x_blk, jnp.ones((_N_BLK, 1), jnp.float32)], axis=1)

    dot = jax.lax.dot_general(
        x_ext, xaug, (((1,), (1,)), ((), ())),
        preferred_element_type=jnp.float32)
    xn2 = jnp.sum(x_blk * x_blk, axis=1, keepdims=True)
    d2 = jnp.maximum(xn2 + dot, 0.0)
    hinge = jnp.maximum(1.0 - jnp.sqrt(d2), 0.0)

    lane = jax.lax.broadcasted_iota(jnp.int32, (_N_BLK, _P), 1)
    mask = pid_ref[...] == (lane + 1)
    seg = jnp.where(mask, d2 - _REP * hinge, 0.0)

    racc[...] += jax.lax.dot_general(
        q_col, hinge, (((0,), (0,)), ((), ())),
        preferred_element_type=jnp.float32)
    tacc[...] += jax.lax.dot_general(
        q_col, seg, (((0,), (0,)), ((), ())),
        preferred_element_type=jnp.float32)

    @pl.when(b == _NB - 1)
    def _final():
        bb = bb_ref[...]
        present = (bb >= 0.0).astype(jnp.float32)
        bbg = jnp.where(bb >= 0.0, bb, 0.0)
        ata = 0.5 * jnp.log((1.0 + bbg) / (1.0 - bbg))
        qa = ata * ata + _Q_MIN
        s = qa * (tacc[...] + _REP * racc[...]) * present
        out_ref[...] = jnp.sum(s, axis=(0, 1), keepdims=True) / _N


@functools.partial(jax.jit)
def _potential_loss(beta, x, particle_id):
    xa, bb = _sc_select(beta, particle_id, x)

    out = pl.pallas_call(
        _dense_kernel,
        grid=(_NB,),
        in_specs=[
            pl.BlockSpec((_N_BLK, 1), lambda b: (b, 0)),
            pl.BlockSpec((_N_BLK, 1), lambda b: (b, 0)),
            pl.BlockSpec((_N_BLK, _D), lambda b: (b, 0)),
            pl.BlockSpec((_P, _D), lambda b: (0, 0)),
            pl.BlockSpec((1, _P), lambda b: (0, 0)),
        ],
        out_specs=pl.BlockSpec((1, 1), lambda b: (0, 0)),
        out_shape=jax.ShapeDtypeStruct((1, 1), jnp.float32),
        scratch_shapes=[
            pltpu.VMEM((1, _P), jnp.float32),
            pltpu.VMEM((1, _P), jnp.float32),
        ],
        compiler_params=pltpu.CompilerParams(
            dimension_semantics=("arbitrary",),
        ),
    )(beta.reshape(_N, 1), particle_id.reshape(_N, 1), x,
      xa, bb.reshape(1, _P))
    return out[0, 0]


def kernel(w, beta, x, y, particle_id):
    return _potential_loss(beta, x, particle_id)


# speedup vs baseline: 1.0174x; 1.0162x over previous
import functools

import jax
import jax.numpy as jnp
from jax import lax
from jax.experimental import pallas as pl
from jax.experimental.pallas import tpu as pltpu
from jax.experimental.pallas import tpu_sc as plsc

_N = 8192
_D = 16
_P = 256
_Q_MIN = 0.01
_REP = 10.0

_NW = 16
_CH = _N // _NW
_PW = _P // _NW
_BIGN = 1 << 30

_N_BLK = 2048
_NB = _N // _N_BLK


@functools.partial(
    pl.kernel,
    mesh=plsc.VectorSubcoreMesh(
        core_axis_name="c", subcore_axis_name="s", num_cores=1),
    compiler_params=pltpu.CompilerParams(needs_layout_passes=False, use_tc_tiling_on_sc=False),
    out_type=[
        jax.ShapeDtypeStruct((_P, _D), jnp.float32),
        jax.ShapeDtypeStruct((_P,), jnp.float32),
    ],
    scratch_types=[
        pltpu.VMEM((_CH,), jnp.float32),
        pltpu.VMEM((_CH,), jnp.int32),
        pltpu.VMEM((_P,), jnp.float32),
        pltpu.VMEM((_P,), jnp.int32),
        pltpu.VMEM_SHARED((_NW, _P), jnp.float32),
        pltpu.VMEM_SHARED((_NW, _P), jnp.int32),
        pltpu.VMEM((_NW, _P), jnp.float32),
        pltpu.VMEM((_NW, _P), jnp.int32),
        pltpu.VMEM((_PW,), jnp.int32),
        pltpu.VMEM((_PW, _D), jnp.float32),
        pltpu.VMEM((_PW,), jnp.float32),
        pltpu.SemaphoreType.DMA,
    ],
)
def _sc_select(beta_hbm, pid_hbm, x_hbm, xa_out, bb_out,
               bbuf, pbuf, tb, tn, sh_b, sh_n, mb, mn, idxv, rows, bbv, sem):
    wid = lax.axis_index("s")
    base = wid * _CH
    lane16 = jnp.arange(16, dtype=jnp.int32)

    pltpu.sync_copy(beta_hbm.at[pl.ds(base, _CH)], bbuf)
    pltpu.sync_copy(pid_hbm.at[pl.ds(base, _CH)], pbuf)

    for k in range(_P // 16):
        tb[pl.ds(k * 16, 16)] = jnp.full((16,), -1.0, jnp.float32)
        tn[pl.ds(k * 16, 16)] = jnp.full((16,), _BIGN, jnp.int32)

    for j in range(_CH // 16):
        bvec = bbuf[pl.ds(j * 16, 16)]
        pvec = pbuf[pl.ds(j * 16, 16)]
        gidx = base + j * 16 + lane16
        valid = pvec >= 1
        pm1 = jnp.where(valid, pvec - 1, 0)

        def _want(_, bvec=bvec, gidx=gidx, valid=valid, pm1=pm1):
            cur = plsc.load_gather(tb, [pm1])
            curn = plsc.load_gather(tn, [pm1])
            better = jnp.logical_or(
                bvec > cur,
                jnp.logical_and(bvec == cur, gidx < curn))
            return jnp.logical_and(valid, better)

        def _body(want, bvec=bvec, gidx=gidx, pm1=pm1, _want=_want):
            plsc.store_scatter(tb, [pm1], bvec, mask=want)
            plsc.store_scatter(tn, [pm1], gidx, mask=want)
            return _want(None)

        lax.while_loop(lambda w: jnp.any(w), _body, _want(None))

    pltpu.sync_copy(tb, sh_b.at[wid])
    pltpu.sync_copy(tn, sh_n.at[wid])
    plsc.subcore_barrier()

    pltpu.sync_copy(sh_b, mb)
    pltpu.sync_copy(sh_n, mn)

    merged_b = jnp.full((16,), -1.0, jnp.float32)
    merged_n = jnp.full((16,), _BIGN, jnp.int32)
    col = wid * _PW
    for t in range(_NW):
        b_t = mb[t, pl.ds(col, _PW)]
        n_t = mn[t, pl.ds(col, _PW)]
        better = jnp.logical_or(
            b_t > merged_b,
            jnp.logical_and(b_t == merged_b, n_t < merged_n))
        merged_b = jnp.where(better, b_t, merged_b)
        merged_n = jnp.where(better, n_t, merged_n)

    idxv[...] = jnp.where(merged_b >= 0.0, merged_n, 0)
    bbv[...] = merged_b
    pltpu.async_copy(x_hbm.at[idxv], rows, sem).wait()

    pltpu.sync_copy(rows, xa_out.at[pl.ds(wid * _PW, _PW)])
    pltpu.sync_copy(bbv, bb_out.at[pl.ds(wid * _PW, _PW)])


def _dense_kernel(beta_ref, pid_ref, x_ref, xa_ref, bb_ref, out_ref, acc):
    b = pl.program_id(0)

    @pl.when(b == 0)
    def _init():
        acc[...] = jnp.zeros((1, _P), jnp.float32)

    beta_col = beta_ref[...]
    at = 0.5 * jnp.log((1.0 + beta_col) / (1.0 - beta_col))
    q_col = at * at + _Q_MIN

    x_blk = x_ref[...]
    xa = xa_ref[...]
    xa2 = jnp.sum(xa * xa, axis=1, keepdims=True)
    xaug = jnp.concatenate(
        [-2.0 * xa, xa2, jnp.ones((_P, 1), jnp.float32)], axis=1)
    xn2 = jnp.sum(x_blk * x_blk, axis=1, keepdims=True)
    x_ext = jnp.concatenate(
        [x_blk, jnp.ones((_N_BLK, 1), jnp.float32), xn2], axis=1)

    d2 = jnp.maximum(jax.lax.dot_general(
        x_ext, xaug, (((1,), (1,)), ((), ())),
        preferred_element_type=jnp.float32), 0.0)
    hinge10 = jnp.maximum(_REP - _REP * jnp.sqrt(d2), 0.0)

    lane = jax.lax.broadcasted_iota(jnp.int32, (_N_BLK, _P), 1)
    mask = pid_ref[...] == (lane + 1)
    comb = jnp.where(mask, d2, hinge10)
    acc[...] += jax.lax.dot_general(
        q_col, comb, (((0,), (0,)), ((), ())),
        preferred_element_type=jnp.float32)

    @pl.when(b == _NB - 1)
    def _final():
        bb = bb_ref[...]
        present = (bb >= 0.0).astype(jnp.float32)
        bbg = jnp.where(bb >= 0.0, bb, 0.0)
        ata = 0.5 * jnp.log((1.0 + bbg) / (1.0 - bbg))
        qa = ata * ata + _Q_MIN
        s = qa * acc[...] * present
        out_ref[...] = jnp.sum(s, axis=(0, 1), keepdims=True) / _N


@functools.partial(jax.jit)
def _potential_loss(beta, x, particle_id):
    xa, bb = _sc_select(beta, particle_id, x)

    out = pl.pallas_call(
        _dense_kernel,
        grid=(_NB,),
        in_specs=[
            pl.BlockSpec((_N_BLK, 1), lambda b: (b, 0)),
            pl.BlockSpec((_N_BLK, 1), lambda b: (b, 0)),
            pl.BlockSpec((_N_BLK, _D), lambda b: (b, 0)),
            pl.BlockSpec((_P, _D), lambda b: (0, 0)),
            pl.BlockSpec((1, _P), lambda b: (0, 0)),
        ],
        out_specs=pl.BlockSpec((1, 1), lambda b: (0, 0)),
        out_shape=jax.ShapeDtypeStruct((1, 1), jnp.float32),
        scratch_shapes=[
            pltpu.VMEM((1, _P), jnp.float32),
        ],
        compiler_params=pltpu.CompilerParams(
            dimension_semantics=("arbitrary",),
        ),
    )(beta.reshape(_N, 1), particle_id.reshape(_N, 1), x,
      xa, bb.reshape(1, _P))
    return out[0, 0]


def kernel(w, beta, x, y, particle_id):
    return _potential_loss(beta, x, particle_id)


# speedup vs baseline: 1.3823x; 1.3587x over previous
import functools

import jax
import jax.numpy as jnp
from jax.experimental import pallas as pl
from jax.experimental.pallas import tpu as pltpu

_N = 8192
_D = 16
_P = 256
_N_BLK = 2048
_NB = _N // _N_BLK
_Q_MIN = 0.01
_REP = 10.0


def _select_kernel(beta_ref, pid_ref, x_ref, q_out, xat_out, bestq_out):
    b = pl.program_id(0)

    @pl.when(b == 0)
    def _init():
        xat_out[...] = jnp.zeros((_D, _P), jnp.float32)
        bestq_out[...] = jnp.full((1, _P), -1.0, jnp.float32)

    beta_col = beta_ref[...]
    at = 0.5 * jnp.log((1.0 + beta_col) / (1.0 - beta_col))
    q_col = at * at + _Q_MIN
    q_out[...] = q_col

    lane = jax.lax.broadcasted_iota(jnp.int32, (_N_BLK, _P), 1)
    mask = pid_ref[...] == (lane + 1)
    n_loc = jax.lax.broadcasted_iota(jnp.int32, (_N_BLK, _P), 0)

    mq = jnp.where(mask, q_col, -1.0)
    bmax = jnp.max(mq, axis=0, keepdims=True)
    nidx = jnp.where(mq == bmax, n_loc, _N)
    bmin = jnp.min(nidx, axis=0, keepdims=True)
    upd = bmax > bestq_out[...]

    sel = jnp.logical_and(n_loc == bmin, upd).astype(jnp.float32)
    xcand = jax.lax.dot_general(
        x_ref[...], sel, (((0,), (0,)), ((), ())),
        preferred_element_type=jnp.float32)
    xat_out[...] = jnp.where(upd, xcand, xat_out[...])
    bestq_out[...] = jnp.where(upd, bmax, bestq_out[...])


def _dense_kernel(q_ref, pid_ref, x_ref, xat_ref, bestq_ref, out_ref, acc):
    b = pl.program_id(0)

    @pl.when(b == 0)
    def _init():
        acc[...] = jnp.zeros((1, _P), jnp.float32)

    q_col = q_ref[...]
    x_blk = x_ref[...]
    xat = xat_ref[...]
    xa2 = jnp.sum(xat * xat, axis=0, keepdims=True)
    xaug = jnp.concatenate(
        [-2.0 * xat, xa2, jnp.ones((1, _P), jnp.float32)], axis=0)
    xn2 = jnp.sum(x_blk * x_blk, axis=1, keepdims=True)
    x_ext = jnp.concatenate(
        [x_blk, jnp.ones((_N_BLK, 1), jnp.float32), xn2], axis=1)

    d2 = jnp.maximum(jax.lax.dot_general(
        x_ext, xaug, (((1,), (0,)), ((), ())),
        preferred_element_type=jnp.float32), 0.0)
    hinge10 = jnp.maximum(_REP - _REP * jnp.sqrt(d2), 0.0)

    lane = jax.lax.broadcasted_iota(jnp.int32, (_N_BLK, _P), 1)
    mask = pid_ref[...] == (lane + 1)
    comb = jnp.where(mask, d2, hinge10)
    acc[...] += jax.lax.dot_general(
        q_col, comb, (((0,), (0,)), ((), ())),
        preferred_element_type=jnp.float32)

    @pl.when(b == _NB - 1)
    def _final():
        bq = bestq_ref[...]
        present = (bq >= 0.0).astype(jnp.float32)
        s = bq * acc[...] * present
        out_ref[...] = jnp.sum(s, axis=(0, 1), keepdims=True) / _N


@functools.partial(jax.jit)
def _potential_loss(beta, x, particle_id):
    beta2 = beta.reshape(_N, 1)
    pid2 = particle_id.reshape(_N, 1)

    q2, xat, bestq = pl.pallas_call(
        _select_kernel,
        grid=(_NB,),
        in_specs=[
            pl.BlockSpec((_N_BLK, 1), lambda b: (b, 0)),
            pl.BlockSpec((_N_BLK, 1), lambda b: (b, 0)),
            pl.BlockSpec((_N_BLK, _D), lambda b: (b, 0)),
        ],
        out_specs=[
            pl.BlockSpec((_N_BLK, 1), lambda b: (b, 0)),
            pl.BlockSpec((_D, _P), lambda b: (0, 0)),
            pl.BlockSpec((1, _P), lambda b: (0, 0)),
        ],
        out_shape=[
            jax.ShapeDtypeStruct((_N, 1), jnp.float32),
            jax.ShapeDtypeStruct((_D, _P), jnp.float32),
            jax.ShapeDtypeStruct((1, _P), jnp.float32),
        ],
        compiler_params=pltpu.CompilerParams(
            dimension_semantics=("arbitrary",),
        ),
    )(beta2, pid2, x)

    out = pl.pallas_call(
        _dense_kernel,
        grid=(_NB,),
        in_specs=[
            pl.BlockSpec((_N_BLK, 1), lambda b: (b, 0)),
            pl.BlockSpec((_N_BLK, 1), lambda b: (b, 0)),
            pl.BlockSpec((_N_BLK, _D), lambda b: (b, 0)),
            pl.BlockSpec((_D, _P), lambda b: (0, 0)),
            pl.BlockSpec((1, _P), lambda b: (0, 0)),
        ],
        out_specs=pl.BlockSpec((1, 1), lambda b: (0, 0)),
        out_shape=jax.ShapeDtypeStruct((1, 1), jnp.float32),
        scratch_shapes=[
            pltpu.VMEM((1, _P), jnp.float32),
        ],
        compiler_params=pltpu.CompilerParams(
            dimension_semantics=("arbitrary",),
        ),
    )(q2, pid2, x, xat, bestq)
    return out[0, 0]


def kernel(w, beta, x, y, particle_id):
    return _potential_loss(beta, x, particle_id)
